# baseline (speedup 1.0000x reference)
import functools

import jax
import jax.numpy as jnp
from jax.experimental import pallas as pl
from jax.experimental.pallas import tpu as pltpu

N_BINS = 10
_LANES = 128
_ROWS_PER_BLOCK = 4000


def _ece_body(nblocks, lo_ref, hi_ref, x_ref, lab_ref, ece_ref, mce_ref,
              acc_scratch):
    i = pl.program_id(0)

    @pl.when(i == 0)
    def _init():
        acc_scratch[...] = jnp.zeros_like(acc_scratch)

    x = x_ref[...]
    r, c = x.shape
    m = jnp.max(x, axis=1, keepdims=True)
    s = jnp.sum(jnp.exp(x - m), axis=1, keepdims=True)
    conf = 1.0 / s

    col = jax.lax.broadcasted_iota(jnp.int32, (r, c), 1)
    pred = jnp.min(jnp.where(x == m, col, c), axis=1, keepdims=True)
    acc = (pred == lab_ref[...]).astype(jnp.float32)

    lo = lo_ref[...]
    hi = hi_ref[...]
    in_bin = ((conf > lo) & (conf <= hi)).astype(jnp.float32)

    cnt = jnp.sum(in_bin, axis=0, keepdims=True)
    sconf = jnp.sum(in_bin * conf, axis=0, keepdims=True)
    sacc = jnp.sum(in_bin * acc, axis=0, keepdims=True)
    acc_scratch[0:1, :] += cnt
    acc_scratch[1:2, :] += sconf
    acc_scratch[2:3, :] += sacc

    @pl.when(i == nblocks - 1)
    def _finish():
        count = acc_scratch[0:1, :]
        sum_conf = acc_scratch[1:2, :]
        sum_acc = acc_scratch[2:3, :]
        total = jnp.float32(nblocks * _ROWS_PER_BLOCK)
        safe = jnp.maximum(count, 1.0)
        gap = jnp.abs(sum_conf / safe - sum_acc / safe)
        nonempty = count > 0.0
        ece_ref[...] = jnp.sum(
            jnp.where(nonempty, gap * (count / total), 0.0),
            axis=1, keepdims=True)
        mce_ref[...] = jnp.max(
            jnp.where(nonempty, gap, -1.0), axis=1, keepdims=True)


def kernel(logits, labels):
    n, c = logits.shape
    assert n % _ROWS_PER_BLOCK == 0
    nblocks = n // _ROWS_PER_BLOCK

    bounds = jnp.linspace(0.0, 1.0, N_BINS + 1).astype(jnp.float32)
    pad = jnp.full((_LANES - N_BINS,), jnp.inf, jnp.float32)
    lo = jnp.concatenate([bounds[:-1], pad]).reshape(1, _LANES)
    hi = jnp.concatenate([bounds[1:], pad]).reshape(1, _LANES)

    labels2 = labels.astype(jnp.int32).reshape(n, 1)

    ece, mce = pl.pallas_call(
        functools.partial(_ece_body, nblocks),
        grid=(nblocks,),
        in_specs=[
            pl.BlockSpec((1, _LANES), lambda i: (0, 0)),
            pl.BlockSpec((1, _LANES), lambda i: (0, 0)),
            pl.BlockSpec((_ROWS_PER_BLOCK, c), lambda i: (i, 0)),
            pl.BlockSpec((_ROWS_PER_BLOCK, 1), lambda i: (i, 0)),
        ],
        out_specs=[
            pl.BlockSpec((1, 1), lambda i: (0, 0)),
            pl.BlockSpec((1, 1), lambda i: (0, 0)),
        ],
        out_shape=[
            jax.ShapeDtypeStruct((1, 1), jnp.float32),
            jax.ShapeDtypeStruct((1, 1), jnp.float32),
        ],
        scratch_shapes=[pltpu.VMEM((8, _LANES), jnp.float32)],
    )(lo, hi, logits, labels2)
    return (ece[0, 0], mce[0, 0])


# speedup vs baseline: 1.0060x; 1.0060x over previous
import functools

import jax
import jax.numpy as jnp
import numpy as np
from jax import lax
from jax.experimental import pallas as pl
from jax.experimental.pallas import tpu as pltpu

N_BINS = 10
_LANES = 128
_R = 4000
_CHUNKS = 2


def _ece_body(nblocks, nrows, lo_ref, hi_ref, x_ref, lab_ref, ece_ref,
              mce_ref, acc_scratch):
    i = pl.program_id(0)

    @pl.when(i == 0)
    def _init():
        acc_scratch[...] = jnp.zeros_like(acc_scratch)

    lo = lo_ref[...]
    hi = hi_ref[...]

    rc = _R // _CHUNKS
    parts = []
    for k in range(_CHUNKS):
        x = x_ref[k * rc:(k + 1) * rc, :]
        lab = lab_ref[k * rc:(k + 1) * rc, :]
        r, c = x.shape
        m = jnp.max(x, axis=1, keepdims=True)
        s = jnp.sum(jnp.exp(x - m), axis=1, keepdims=True)
        conf = 1.0 / s

        col = lax.broadcasted_iota(jnp.int32, (r, c), 1)
        hit = (x == m) & (col == lab)
        acc = jnp.max(hit.astype(jnp.float32), axis=1, keepdims=True)

        in_bin = ((conf > lo) & (conf <= hi)).astype(jnp.float32)
        w = jnp.concatenate(
            [jnp.ones_like(conf), conf, acc,
             jnp.zeros((r, 5), jnp.float32)], axis=1)
        parts.append(lax.dot_general(
            w, in_bin, (((0,), (0,)), ((), ())),
            preferred_element_type=jnp.float32))
    acc_scratch[...] += functools.reduce(lambda a, b: a + b, parts)

    @pl.when(i == nblocks - 1)
    def _finish():
        count = acc_scratch[0:1, :]
        sum_conf = acc_scratch[1:2, :]
        sum_acc = acc_scratch[2:3, :]
        total = jnp.float32(nrows)
        safe = jnp.maximum(count, 1.0)
        gap = jnp.abs(sum_conf / safe - sum_acc / safe)
        nonempty = count > 0.0
        ece_ref[...] = jnp.sum(
            jnp.where(nonempty, gap * (count / total), 0.0),
            axis=1, keepdims=True)
        mce_ref[...] = jnp.max(
            jnp.where(nonempty, gap, -1.0), axis=1, keepdims=True)


def kernel(logits, labels):
    n, c = logits.shape
    assert n % _R == 0
    nblocks = n // _R

    bounds = np.linspace(0.0, 1.0, N_BINS + 1).astype(np.float32)
    lo = jnp.asarray(np.concatenate(
        [bounds[:-1], np.full(_LANES - N_BINS, np.inf, np.float32)]
    ).reshape(1, _LANES))
    hi = jnp.asarray(np.concatenate(
        [bounds[1:], np.full(_LANES - N_BINS, np.inf, np.float32)]
    ).reshape(1, _LANES))

    labels2 = labels.astype(jnp.int32).reshape(n, 1)

    ece, mce = pl.pallas_call(
        functools.partial(_ece_body, nblocks, n),
        grid=(nblocks,),
        in_specs=[
            pl.BlockSpec((1, _LANES), lambda i: (0, 0)),
            pl.BlockSpec((1, _LANES), lambda i: (0, 0)),
            pl.BlockSpec((_R, c), lambda i: (i, 0)),
            pl.BlockSpec((_R, 1), lambda i: (i, 0)),
        ],
        out_specs=[
            pl.BlockSpec((1, 1), lambda i: (0, 0)),
            pl.BlockSpec((1, 1), lambda i: (0, 0)),
        ],
        out_shape=[
            jax.ShapeDtypeStruct((1, 1), jnp.float32),
            jax.ShapeDtypeStruct((1, 1), jnp.float32),
        ],
        scratch_shapes=[pltpu.VMEM((8, _LANES), jnp.float32)],
    )(lo, hi, logits, labels2)
    return (ece[0, 0], mce[0, 0])


# speedup vs baseline: 1.3271x; 1.3192x over previous
import functools

import jax
import jax.numpy as jnp
import numpy as np
from jax import lax
from jax.experimental import pallas as pl
from jax.experimental.pallas import tpu as pltpu

N_BINS = 10
_LANES = 128
_R = 8000
_LW = 125
_LC = _R // _LW


def _ece_body(nblocks, nrows, lo_ref, hi_ref, x_ref, lab_ref, ece_ref,
              mce_ref, acc_scratch):
    i = pl.program_id(0)

    @pl.when(i == 0)
    def _init():
        acc_scratch[...] = jnp.zeros_like(acc_scratch)

    lo = lo_ref[...]
    hi = hi_ref[...]

    x = x_ref[...]
    r, c = x.shape
    ones_c = jnp.ones((c, 1), jnp.float32)

    m = jnp.max(x, axis=1, keepdims=True)
    e = jnp.exp(x - m)
    s1 = lax.dot_general(e, ones_c, (((1,), (0,)), ((), ())),
                         preferred_element_type=jnp.float32)
    conf = 1.0 / s1

    eye = jnp.eye(_LW, dtype=jnp.float32)
    lab_d = lab_ref[...]
    lab_cols = [
        lax.dot_general(eye, lab_d[k:k + 1, :], (((1,), (1,)), ((), ())),
                        preferred_element_type=jnp.float32)
        for k in range(_LC)
    ]
    lab = jnp.concatenate(lab_cols, axis=0)

    col = lax.broadcasted_iota(jnp.int32, (r, c), 1).astype(jnp.float32)
    hit = ((x == m) & (col == lab)).astype(jnp.float32)
    acc = lax.dot_general(hit, ones_c, (((1,), (0,)), ((), ())),
                          preferred_element_type=jnp.float32)

    in_bin = ((conf > lo) & (conf <= hi)).astype(jnp.float32)
    dn = (((0,), (0,)), ((), ()))
    cnt = lax.dot_general(jnp.ones_like(conf), in_bin, dn,
                          preferred_element_type=jnp.float32)
    sconf = lax.dot_general(conf, in_bin, dn,
                            preferred_element_type=jnp.float32)
    sacc = lax.dot_general(acc, in_bin, dn,
                           preferred_element_type=jnp.float32)
    acc_scratch[0:1, :] += cnt
    acc_scratch[1:2, :] += sconf
    acc_scratch[2:3, :] += sacc

    @pl.when(i == nblocks - 1)
    def _finish():
        count = acc_scratch[0:1, :]
        sum_conf = acc_scratch[1:2, :]
        sum_acc = acc_scratch[2:3, :]
        total = jnp.float32(nrows)
        safe = jnp.maximum(count, 1.0)
        gap = jnp.abs(sum_conf / safe - sum_acc / safe)
        nonempty = count > 0.0
        ece_ref[...] = jnp.sum(
            jnp.where(nonempty, gap * (count / total), 0.0),
            axis=1, keepdims=True)
        mce_ref[...] = jnp.max(
            jnp.where(nonempty, gap, -1.0), axis=1, keepdims=True)


def kernel(logits, labels):
    n, c = logits.shape
    assert n % _R == 0
    nblocks = n // _R

    bounds = np.linspace(0.0, 1.0, N_BINS + 1).astype(np.float32)
    lo = jnp.asarray(np.concatenate(
        [bounds[:-1], np.full(_LANES - N_BINS, np.inf, np.float32)]
    ).reshape(1, _LANES))
    hi = jnp.asarray(np.concatenate(
        [bounds[1:], np.full(_LANES - N_BINS, np.inf, np.float32)]
    ).reshape(1, _LANES))

    labels2 = labels.astype(jnp.float32).reshape(n // _LW, _LW)

    ece, mce = pl.pallas_call(
        functools.partial(_ece_body, nblocks, n),
        grid=(nblocks,),
        in_specs=[
            pl.BlockSpec((1, _LANES), lambda i: (0, 0)),
            pl.BlockSpec((1, _LANES), lambda i: (0, 0)),
            pl.BlockSpec((_R, c), lambda i: (i, 0)),
            pl.BlockSpec((_LC, _LW), lambda i: (i, 0)),
        ],
        out_specs=[
            pl.BlockSpec((1, 1), lambda i: (0, 0)),
            pl.BlockSpec((1, 1), lambda i: (0, 0)),
        ],
        out_shape=[
            jax.ShapeDtypeStruct((1, 1), jnp.float32),
            jax.ShapeDtypeStruct((1, 1), jnp.float32),
        ],
        scratch_shapes=[pltpu.VMEM((8, _LANES), jnp.float32)],
    )(lo, hi, logits, labels2)
    return (ece[0, 0], mce[0, 0])


# speedup vs baseline: 1.3374x; 1.0078x over previous
import functools

import jax
import jax.numpy as jnp
import numpy as np
from jax import lax
from jax.experimental import pallas as pl
from jax.experimental.pallas import tpu as pltpu

N_BINS = 10
_LANES = 128
_R = 20000
_LW = 125
_LC = _R // _LW


def _ece_body(nblocks, nrows, lo_ref, hi_ref, x_ref, lab_ref, ece_ref,
              mce_ref, acc_scratch):
    i = pl.program_id(0)

    @pl.when(i == 0)
    def _init():
        acc_scratch[...] = jnp.zeros_like(acc_scratch)

    lo = lo_ref[...]
    hi = hi_ref[...]

    x = x_ref[...]
    r, c = x.shape
    ones_c = jnp.ones((c, 1), jnp.float32)

    m = jnp.max(x, axis=1, keepdims=True)
    e = jnp.exp(x - m)
    s1 = lax.dot_general(e, ones_c, (((1,), (0,)), ((), ())),
                         preferred_element_type=jnp.float32)
    conf = 1.0 / s1

    eye = jnp.eye(_LW, dtype=jnp.float32)
    lab_d = lab_ref[...]
    lab_cols = [
        lax.dot_general(eye, lab_d[k:k + 1, :], (((1,), (1,)), ((), ())),
                        preferred_element_type=jnp.float32)
        for k in range(_LC)
    ]
    lab = jnp.concatenate(lab_cols, axis=0)

    col = lax.broadcasted_iota(jnp.int32, (r, c), 1).astype(jnp.float32)
    hit = ((x == m) & (col == lab)).astype(jnp.float32)
    acc = lax.dot_general(hit, ones_c, (((1,), (0,)), ((), ())),
                          preferred_element_type=jnp.float32)

    in_bin = ((conf > lo) & (conf <= hi)).astype(jnp.float32)
    dn = (((0,), (0,)), ((), ()))
    cnt = lax.dot_general(jnp.ones_like(conf), in_bin, dn,
                          preferred_element_type=jnp.float32)
    sconf = lax.dot_general(conf, in_bin, dn,
                            preferred_element_type=jnp.float32)
    sacc = lax.dot_general(acc, in_bin, dn,
                           preferred_element_type=jnp.float32)
    acc_scratch[0:1, :] += cnt
    acc_scratch[1:2, :] += sconf
    acc_scratch[2:3, :] += sacc

    @pl.when(i == nblocks - 1)
    def _finish():
        count = acc_scratch[0:1, :]
        sum_conf = acc_scratch[1:2, :]
        sum_acc = acc_scratch[2:3, :]
        total = jnp.float32(nrows)
        safe = jnp.maximum(count, 1.0)
        gap = jnp.abs(sum_conf / safe - sum_acc / safe)
        nonempty = count > 0.0
        ece_ref[...] = jnp.sum(
            jnp.where(nonempty, gap * (count / total), 0.0),
            axis=1, keepdims=True)
        mce_ref[...] = jnp.max(
            jnp.where(nonempty, gap, -1.0), axis=1, keepdims=True)


def kernel(logits, labels):
    n, c = logits.shape
    assert n % _R == 0
    nblocks = n // _R

    bounds = np.linspace(0.0, 1.0, N_BINS + 1).astype(np.float32)
    lo = jnp.asarray(np.concatenate(
        [bounds[:-1], np.full(_LANES - N_BINS, np.inf, np.float32)]
    ).reshape(1, _LANES))
    hi = jnp.asarray(np.concatenate(
        [bounds[1:], np.full(_LANES - N_BINS, np.inf, np.float32)]
    ).reshape(1, _LANES))

    labels2 = labels.astype(jnp.float32).reshape(n // _LW, _LW)

    ece, mce = pl.pallas_call(
        functools.partial(_ece_body, nblocks, n),
        grid=(nblocks,),
        in_specs=[
            pl.BlockSpec((1, _LANES), lambda i: (0, 0)),
            pl.BlockSpec((1, _LANES), lambda i: (0, 0)),
            pl.BlockSpec((_R, c), lambda i: (i, 0)),
            pl.BlockSpec((_LC, _LW), lambda i: (i, 0)),
        ],
        out_specs=[
            pl.BlockSpec((1, 1), lambda i: (0, 0)),
            pl.BlockSpec((1, 1), lambda i: (0, 0)),
        ],
        out_shape=[
            jax.ShapeDtypeStruct((1, 1), jnp.float32),
            jax.ShapeDtypeStruct((1, 1), jnp.float32),
        ],
        scratch_shapes=[pltpu.VMEM((8, _LANES), jnp.float32)],
    )(lo, hi, logits, labels2)
    return (ece[0, 0], mce[0, 0])


# speedup vs baseline: 1.3376x; 1.0001x over previous
import functools

import jax
import jax.numpy as jnp
import numpy as np
from jax import lax
from jax.experimental import pallas as pl
from jax.experimental.pallas import tpu as pltpu

N_BINS = 10
_LANES = 128
_R = 20000
_LW = 125
_LC = _R // _LW


def _ece_body(nblocks, nrows, lo_ref, hi_ref, x_ref, lab_ref, ece_ref,
              mce_ref, acc_scratch):
    i = pl.program_id(0)

    @pl.when(i == 0)
    def _init():
        acc_scratch[...] = jnp.zeros_like(acc_scratch)

    lo = lo_ref[...]
    hi = hi_ref[...]

    x = x_ref[...]
    r, c = x.shape
    ones_c = jnp.ones((c, 1), jnp.float32)

    m = jnp.max(x, axis=1, keepdims=True)
    e = jnp.exp(x - m)
    s1 = lax.dot_general(e, ones_c, (((1,), (0,)), ((), ())),
                         preferred_element_type=jnp.float32)
    conf = 1.0 / s1

    eye = jnp.eye(_LW, dtype=jnp.float32)
    lab_d = lab_ref[...]
    lab_cols = [
        lax.dot_general(eye, lab_d[k:k + 1, :], (((1,), (1,)), ((), ())),
                        preferred_element_type=jnp.float32)
        for k in range(_LC)
    ]
    lab = jnp.concatenate(lab_cols, axis=0)

    col1 = lax.broadcasted_iota(jnp.int32, (1, c), 1).astype(jnp.float32)
    col = jnp.broadcast_to(col1, (r, c))
    hit = ((x == m) & (col == lab)).astype(jnp.float32)
    acc = lax.dot_general(hit, ones_c, (((1,), (0,)), ((), ())),
                          preferred_element_type=jnp.float32)

    in_bin = ((conf > lo) & (conf <= hi)).astype(jnp.float32)
    dn = (((0,), (0,)), ((), ()))
    cnt = lax.dot_general(jnp.ones_like(conf), in_bin, dn,
                          preferred_element_type=jnp.float32)
    sconf = lax.dot_general(conf, in_bin, dn,
                            preferred_element_type=jnp.float32)
    sacc = lax.dot_general(acc, in_bin, dn,
                           preferred_element_type=jnp.float32)
    acc_scratch[0:1, :] += cnt
    acc_scratch[1:2, :] += sconf
    acc_scratch[2:3, :] += sacc

    @pl.when(i == nblocks - 1)
    def _finish():
        count = acc_scratch[0:1, :]
        sum_conf = acc_scratch[1:2, :]
        sum_acc = acc_scratch[2:3, :]
        total = jnp.float32(nrows)
        safe = jnp.maximum(count, 1.0)
        gap = jnp.abs(sum_conf / safe - sum_acc / safe)
        nonempty = count > 0.0
        ece_ref[...] = jnp.sum(
            jnp.where(nonempty, gap * (count / total), 0.0),
            axis=1, keepdims=True)
        mce_ref[...] = jnp.max(
            jnp.where(nonempty, gap, -1.0), axis=1, keepdims=True)


def kernel(logits, labels):
    n, c = logits.shape
    assert n % _R == 0
    nblocks = n // _R

    bounds = np.linspace(0.0, 1.0, N_BINS + 1).astype(np.float32)
    lo = jnp.asarray(np.concatenate(
        [bounds[:-1], np.full(_LANES - N_BINS, np.inf, np.float32)]
    ).reshape(1, _LANES))
    hi = jnp.asarray(np.concatenate(
        [bounds[1:], np.full(_LANES - N_BINS, np.inf, np.float32)]
    ).reshape(1, _LANES))

    labels2 = labels.astype(jnp.float32).reshape(n // _LW, _LW)

    ece, mce = pl.pallas_call(
        functools.partial(_ece_body, nblocks, n),
        grid=(nblocks,),
        in_specs=[
            pl.BlockSpec((1, _LANES), lambda i: (0, 0)),
            pl.BlockSpec((1, _LANES), lambda i: (0, 0)),
            pl.BlockSpec((_R, c), lambda i: (i, 0)),
            pl.BlockSpec((_LC, _LW), lambda i: (i, 0)),
        ],
        out_specs=[
            pl.BlockSpec((1, 1), lambda i: (0, 0)),
            pl.BlockSpec((1, 1), lambda i: (0, 0)),
        ],
        out_shape=[
            jax.ShapeDtypeStruct((1, 1), jnp.float32),
            jax.ShapeDtypeStruct((1, 1), jnp.float32),
        ],
        scratch_shapes=[pltpu.VMEM((8, _LANES), jnp.float32)],
    )(lo, hi, logits, labels2)
    return (ece[0, 0], mce[0, 0])
